# Initial kernel scaffold; baseline (speedup 1.0000x reference)
import jax
import jax.numpy as jnp
from jax.experimental import pallas as pl


def kernel(point_normalized, point_xyz, point_offset, point_label, gt_plane_center, gt_plane_center_normalized, gt_center_sem_cls_label, gt_plane_present):
    raise NotImplementedError("write your pallas kernel here")


# speedup vs baseline: 1.0486x; 1.0486x over previous
import functools

import jax
import jax.numpy as jnp
from jax import lax
from jax.experimental import pallas as pl
from jax.experimental.pallas import tpu as pltpu
from jax.experimental.pallas import tpu_sc as plsc

B, N, G, C = 8, 4096, 64, 64
LANES = 16
SUBCORES = 32
PTS_PER_W = (B * N) // SUBCORES
VECS_PER_W = PTS_PER_W // LANES
QUARTERS = SUBCORES // B
WD = 1.0
WA = 1.0


def _rsqrt_nr(x):
    i = lax.bitcast_convert_type(x, jnp.int32)
    i = jnp.int32(0x5F3759DF) - (i >> 1)
    y = lax.bitcast_convert_type(i, jnp.float32)
    for _ in range(3):
        y = y * (1.5 - 0.5 * x * y * y)
    return y


def _sqrt(x):
    return x * _rsqrt_nr(x)


def _sc_body(pts_hbm, off_hbm, lbl_hbm, ctr_hbm, plab_hbm, pres_hbm,
             pd_hbm, pa_hbm,
             pxv, pyv, pzv, oxv, oyv, ozv, lblv,
             cxv, cyv, czv, plabv, presv, cntv, idsv, accd, acca):
    wid = lax.axis_index("s") * 2 + lax.axis_index("c")
    b = wid // QUARTERS
    n0 = (wid % QUARTERS) * PTS_PER_W
    pbase = b * 3 * N + n0
    cbase = b * 3 * G

    pltpu.sync_copy(pts_hbm.at[pl.ds(pbase, PTS_PER_W)], pxv)
    pltpu.sync_copy(pts_hbm.at[pl.ds(pbase + N, PTS_PER_W)], pyv)
    pltpu.sync_copy(pts_hbm.at[pl.ds(pbase + 2 * N, PTS_PER_W)], pzv)
    pltpu.sync_copy(off_hbm.at[pl.ds(pbase, PTS_PER_W)], oxv)
    pltpu.sync_copy(off_hbm.at[pl.ds(pbase + N, PTS_PER_W)], oyv)
    pltpu.sync_copy(off_hbm.at[pl.ds(pbase + 2 * N, PTS_PER_W)], ozv)
    pltpu.sync_copy(lbl_hbm.at[pl.ds(b * N + n0, PTS_PER_W)], lblv)
    pltpu.sync_copy(ctr_hbm.at[pl.ds(cbase, G)], cxv)
    pltpu.sync_copy(ctr_hbm.at[pl.ds(cbase + G, G)], cyv)
    pltpu.sync_copy(ctr_hbm.at[pl.ds(cbase + 2 * G, G)], czv)
    pltpu.sync_copy(plab_hbm.at[pl.ds(b * G, G)], plabv)
    pltpu.sync_copy(pres_hbm.at[pl.ds(b * G, G)], presv)

    lane = lax.iota(jnp.int32, LANES)
    lane0 = lane == 0
    zero16 = jnp.zeros((LANES,), jnp.int32)
    for i in range(C // LANES):
        cntv[pl.ds(i * LANES, LANES)] = zero16
    for g in range(G):
        gsplat = jnp.full((LANES,), g, jnp.int32)
        lg = plsc.load_gather(plabv, [gsplat])
        kg = plsc.load_gather(cntv, [lg])
        plsc.store_scatter(idsv, [lg * G + kg], gsplat, mask=lane0)
        plsc.store_scatter(cntv, [lg], kg + 1, mask=lane0)

    eps = jnp.float32(1e-10)

    def point_vec(v, carry):
        ad, aa = carry
        s = pl.ds(v * LANES, LANES)
        px, py, pz = pxv[s], pyv[s], pzv[s]
        ox, oy, oz = oxv[s], oyv[s], ozv[s]
        lbl = lblv[s]
        ofn = _sqrt(ox * ox + oy * oy + oz * oz)
        r_off = 1.0 / (ofn + eps)
        kcount = plsc.load_gather(cntv, [lbl])
        kmax = jnp.max(kcount)

        def pair(k, carry2):
            ad2, aa2 = carry2
            kk = jnp.full((LANES,), k, jnp.int32)
            valid = kk < kcount
            gid_raw = plsc.load_gather(idsv, [lbl * G + kk])
            gid = jnp.where(valid, gid_raw, 0)
            cx = plsc.load_gather(cxv, [gid])
            cy = plsc.load_gather(cyv, [gid])
            cz = plsc.load_gather(czv, [gid])
            w = jnp.where(valid, plsc.load_gather(presv, [gid]),
                          jnp.float32(0.0))
            cvx, cvy, cvz = cx - px, cy - py, cz - pz
            dist = (jnp.abs(ox - cvx) + jnp.abs(oy - cvy)
                    + jnp.abs(oz - cvz))
            n2 = cvx * cvx + cvy * cvy + cvz * cvz
            dot = ox * cvx + oy * cvy + oz * cvz
            ang = -(dot * r_off) / (_sqrt(n2) + eps)
            return ad2 + dist * w, aa2 + ang * w

        return lax.fori_loop(0, kmax, pair, (ad, aa))

    zf = jnp.zeros((LANES,), jnp.float32)
    ad, aa = lax.fori_loop(0, VECS_PER_W, point_vec, (zf, zf))
    accd[...] = ad
    acca[...] = aa
    pltpu.sync_copy(accd, pd_hbm.at[pl.ds(wid * LANES, LANES)])
    pltpu.sync_copy(acca, pa_hbm.at[pl.ds(wid * LANES, LANES)])


_sc_kernel = functools.partial(
    pl.kernel,
    out_type=[
        jax.ShapeDtypeStruct((SUBCORES * LANES,), jnp.float32),
        jax.ShapeDtypeStruct((SUBCORES * LANES,), jnp.float32),
    ],
    mesh=plsc.VectorSubcoreMesh(core_axis_name="c", subcore_axis_name="s"),
    compiler_params=pltpu.CompilerParams(needs_layout_passes=False),
    scratch_types=[
        pltpu.VMEM((PTS_PER_W,), jnp.float32),
        pltpu.VMEM((PTS_PER_W,), jnp.float32),
        pltpu.VMEM((PTS_PER_W,), jnp.float32),
        pltpu.VMEM((PTS_PER_W,), jnp.float32),
        pltpu.VMEM((PTS_PER_W,), jnp.float32),
        pltpu.VMEM((PTS_PER_W,), jnp.float32),
        pltpu.VMEM((PTS_PER_W,), jnp.int32),
        pltpu.VMEM((G,), jnp.float32),
        pltpu.VMEM((G,), jnp.float32),
        pltpu.VMEM((G,), jnp.float32),
        pltpu.VMEM((G,), jnp.int32),
        pltpu.VMEM((G,), jnp.float32),
        pltpu.VMEM((C,), jnp.int32),
        pltpu.VMEM((C * G,), jnp.int32),
        pltpu.VMEM((LANES,), jnp.float32),
        pltpu.VMEM((LANES,), jnp.float32),
    ],
)(_sc_body)


def _combine_body(pd_ref, pa_ref, of_ref, od_ref, oa_ref):
    scale = jnp.float32(1.0 / (N * B))
    sd = jnp.sum(pd_ref[...]) * (WD * scale)
    sa = jnp.sum(pa_ref[...]) * (WA * scale)
    of_ref[...] = jnp.full((1, 1), sd + sa, jnp.float32)
    od_ref[...] = jnp.full((1, 1), sd, jnp.float32)
    oa_ref[...] = jnp.full((1, 1), sa, jnp.float32)


_combine = pl.pallas_call(
    _combine_body,
    out_shape=[
        jax.ShapeDtypeStruct((1, 1), jnp.float32),
        jax.ShapeDtypeStruct((1, 1), jnp.float32),
        jax.ShapeDtypeStruct((1, 1), jnp.float32),
    ],
)


def kernel(point_normalized, point_xyz, point_offset, point_label,
           gt_plane_center, gt_plane_center_normalized,
           gt_center_sem_cls_label, gt_plane_present):
    ptsT = jnp.transpose(point_xyz[..., :3], (0, 2, 1)).reshape(-1)
    offT = jnp.transpose(point_offset[..., :3], (0, 2, 1)).reshape(-1)
    ctrT = jnp.transpose(
        gt_plane_center_normalized[..., :3], (0, 2, 1)).reshape(-1)
    lbl = point_label.astype(jnp.int32).reshape(-1)
    plab = gt_center_sem_cls_label.astype(jnp.int32).reshape(-1)
    pres = gt_plane_present.astype(jnp.float32).reshape(-1)
    pd, pa = _sc_kernel(ptsT, offT, lbl, ctrT, plab, pres)
    f, d, a = _combine(pd, pa)
    return f[0, 0], d[0, 0], a[0, 0]
